# Initial kernel scaffold; baseline (speedup 1.0000x reference)
import jax
import jax.numpy as jnp
from jax.experimental import pallas as pl


def kernel(x, edge_index, W1l, b1, W1r, W2l, b2, W2r, W3l, b3, W3r):
    raise NotImplementedError("write your pallas kernel here")


# speedup vs baseline: 2.9124x; 2.9124x over previous
import functools

import jax
import jax.numpy as jnp
from jax import lax
from jax.experimental import pallas as pl
from jax.experimental.pallas import tpu as pltpu
from jax.experimental.pallas import tpu_sc as plsc

N = 10000
D = 128
NC = 2
NS = 16
NW = NC * NS
K = 128
ROWS_PER_TILE = 640
N_PAD = NS * ROWS_PER_TILE
IB = 8


def _sc_agg_kernel(ch):
    mesh = plsc.VectorSubcoreMesh(core_axis_name="c", subcore_axis_name="s")

    def body(y_hbm, srcs_hbm, dsts_hbm, agg_hbm, idx_s, idx_d, rows, agg_sh,
             sem):
        c = lax.axis_index("c")
        s = lax.axis_index("s")
        wid = c * NS + s
        base = s * ROWS_PER_TILE

        @pl.loop(0, K)
        def _zfill(i):
            for cc in range(D // 16):
                rows[i, pl.ds(cc * 16, 16)] = jnp.zeros((16,), jnp.float32)
        for k in range(ROWS_PER_TILE // K):
            pltpu.sync_copy(rows, agg_sh.at[pl.ds(base + k * K, K)])
        plsc.subcore_barrier()

        @pl.loop(0, ch // IB)
        def _outer(o):
            pltpu.sync_copy(srcs_hbm.at[wid, pl.ds(o * IB, IB)], idx_s)
            pltpu.sync_copy(dsts_hbm.at[wid, pl.ds(o * IB, IB)], idx_d)
            for j in range(IB):
                pltpu.async_copy(y_hbm.at[idx_s.at[j]], rows, sem).wait()
                pltpu.sync_copy(rows, agg_sh.at[idx_d.at[j]], add=True)
        plsc.subcore_barrier()

        for k in range(ROWS_PER_TILE // K):
            pltpu.sync_copy(agg_sh.at[pl.ds(base + k * K, K)], rows)
            pltpu.sync_copy(
                rows, agg_hbm.at[pl.ds(c * N_PAD + base + k * K, K)])

    return pl.kernel(
        body,
        out_type=jax.ShapeDtypeStruct((NC * N_PAD, D), jnp.float32),
        mesh=mesh,
        scratch_types=[
            pltpu.VMEM((IB, K), jnp.int32),
            pltpu.VMEM((IB, K), jnp.int32),
            pltpu.VMEM((K, D), jnp.float32),
            pltpu.VMEM_SHARED((N_PAD, D), jnp.float32),
            pltpu.SemaphoreType.DMA,
        ],
        name="sc_segment_sum",
    )


def _sc_deg_kernel(ch):
    mesh = plsc.VectorSubcoreMesh(core_axis_name="c", subcore_axis_name="s")

    def body(dsts_hbm, degw_hbm, idx_d, rows, degw_sh):
        c = lax.axis_index("c")
        s = lax.axis_index("s")
        wid = c * NS + s
        base = s * ROWS_PER_TILE

        @pl.loop(0, K)
        def _zfill(i):
            for cc in range(D // 16):
                rows[i, pl.ds(cc * 16, 16)] = jnp.zeros((16,), jnp.float32)
        for k in range(ROWS_PER_TILE // K):
            pltpu.sync_copy(rows, degw_sh.at[pl.ds(base + k * K, K)])

        @pl.loop(0, K)
        def _ofill(i):
            for cc in range(D // 16):
                rows[i, pl.ds(cc * 16, 16)] = jnp.ones((16,), jnp.float32)
        plsc.subcore_barrier()

        @pl.loop(0, ch // IB)
        def _outer(o):
            pltpu.sync_copy(dsts_hbm.at[wid, pl.ds(o * IB, IB)], idx_d)
            for j in range(IB):
                pltpu.sync_copy(rows, degw_sh.at[idx_d.at[j]], add=True)
        plsc.subcore_barrier()

        for k in range(ROWS_PER_TILE // K):
            pltpu.sync_copy(degw_sh.at[pl.ds(base + k * K, K)], rows)
            pltpu.sync_copy(
                rows, degw_hbm.at[pl.ds(c * N_PAD + base + k * K, K)])

    return pl.kernel(
        body,
        out_type=jax.ShapeDtypeStruct((NC * N_PAD, D), jnp.float32),
        mesh=mesh,
        scratch_types=[
            pltpu.VMEM((IB, K), jnp.int32),
            pltpu.VMEM((K, D), jnp.float32),
            pltpu.VMEM_SHARED((N_PAD, D), jnp.float32),
        ],
        name="sc_degree_count",
    )


def _transform_body(h_ref, wl_ref, wr_ref, b_ref, y_ref, z_ref):
    hb = h_ref[...]
    y_ref[...] = jnp.dot(hb, wl_ref[...], preferred_element_type=jnp.float32)
    z_ref[...] = (
        jnp.dot(hb, wr_ref[...], preferred_element_type=jnp.float32)
        + b_ref[...]
    )


def _transform(h, wl, wr, b):
    r = 2000
    return pl.pallas_call(
        _transform_body,
        grid=(N // r,),
        in_specs=[
            pl.BlockSpec((r, D), lambda i: (i, 0)),
            pl.BlockSpec((D, D), lambda i: (0, 0)),
            pl.BlockSpec((D, D), lambda i: (0, 0)),
            pl.BlockSpec((1, D), lambda i: (0, 0)),
        ],
        out_specs=[
            pl.BlockSpec((r, D), lambda i: (i, 0)),
            pl.BlockSpec((r, D), lambda i: (i, 0)),
        ],
        out_shape=[jax.ShapeDtypeStruct((N, D), jnp.float32)] * 2,
    )(h, wl, wr, b.reshape(1, D))


def _combine_body(a_ref, dw_ref, z_ref, o_ref, *, relu):
    deg = dw_ref[0, :, 0:1] + dw_ref[1, :, 0:1]
    recip = 1.0 / jnp.maximum(deg, 1.0)
    o = (a_ref[0] + a_ref[1]) * recip + z_ref[...]
    o_ref[...] = jnp.maximum(o, 0.0) if relu else o


def _combine(agg, degw, z, relu):
    r = 1000
    a3 = agg.reshape(NC, N_PAD, D)
    d3 = degw.reshape(NC, N_PAD, D)
    return pl.pallas_call(
        functools.partial(_combine_body, relu=relu),
        grid=(N // r,),
        in_specs=[
            pl.BlockSpec((NC, r, D), lambda i: (0, i, 0)),
            pl.BlockSpec((NC, r, D), lambda i: (0, i, 0)),
            pl.BlockSpec((r, D), lambda i: (i, 0)),
        ],
        out_specs=pl.BlockSpec((r, D), lambda i: (i, 0)),
        out_shape=jax.ShapeDtypeStruct((N, D), jnp.float32),
    )(a3, d3, z)


def kernel(x, edge_index, W1l, b1, W1r, W2l, b2, W2r, W3l, b3, W3r):
    e = edge_index.shape[1]
    ch = -(-e // (NW * K))
    ch = -(-ch // IB) * IB
    e_pad = NW * ch * K
    src = edge_index[0].astype(jnp.int32)
    dst = edge_index[1].astype(jnp.int32)
    pad = e_pad - e
    srcs = jnp.concatenate([src, jnp.zeros((pad,), jnp.int32)])
    dsts = jnp.concatenate([dst, jnp.full((pad,), N, jnp.int32)])
    srcs = srcs.reshape(NW, ch, K)
    dsts = dsts.reshape(NW, ch, K)

    agg_k = _sc_agg_kernel(ch)
    degw = _sc_deg_kernel(ch)(dsts)

    y1, z1 = _transform(x, W1l, W1r, b1)
    a1 = agg_k(y1, srcs, dsts)
    h1 = _combine(a1, degw, z1, True)

    y2, z2 = _transform(h1, W2l, W2r, b2)
    a2 = agg_k(y2, srcs, dsts)
    h2 = _combine(a2, degw, z2, True)

    y3, z3 = _transform(h2, W3l, W3r, b3)
    a3 = agg_k(y3, srcs, dsts)
    return _combine(a3, degw, z3, False)


# speedup vs baseline: 9.6150x; 3.3014x over previous
import functools

import jax
import jax.numpy as jnp
from jax import lax
from jax.experimental import pallas as pl
from jax.experimental.pallas import tpu as pltpu
from jax.experimental.pallas import tpu_sc as plsc

N = 10000
D = 128
NC = 2
NS = 16
NW = NC * NS
K = 128
ROWS_PER_TILE = 640
N_PAD = NS * ROWS_PER_TILE
IB = 8


def _sc_agg_kernel(ch):
    mesh = plsc.VectorSubcoreMesh(core_axis_name="c", subcore_axis_name="s")

    def body(y_hbm, srcs_hbm, dsts_hbm, agg_hbm, idx_s, idx_d, rows0, rows1,
             agg_sh, sem0, sem1):
        c = lax.axis_index("c")
        s = lax.axis_index("s")
        wid = c * NS + s
        base = s * ROWS_PER_TILE

        @pl.loop(0, K)
        def _zfill(i):
            for cc in range(D // 16):
                rows0[i, pl.ds(cc * 16, 16)] = jnp.zeros((16,), jnp.float32)
        for k in range(ROWS_PER_TILE // K):
            pltpu.sync_copy(rows0, agg_sh.at[pl.ds(base + k * K, K)])
        plsc.subcore_barrier()

        @pl.loop(0, ch // IB)
        def _outer(o):
            pltpu.sync_copy(srcs_hbm.at[wid, pl.ds(o * IB, IB)], idx_s)
            pltpu.sync_copy(dsts_hbm.at[wid, pl.ds(o * IB, IB)], idx_d)
            pltpu.async_copy(y_hbm.at[idx_s.at[0]], rows0, sem0)
            for j in range(IB):
                cur, csem = (rows0, sem0) if j % 2 == 0 else (rows1, sem1)
                if j + 1 < IB:
                    nbuf, nsem = (rows1, sem1) if j % 2 == 0 else (rows0,
                                                                   sem0)
                    pltpu.async_copy(y_hbm.at[idx_s.at[j + 1]], nbuf, nsem)
                pltpu.make_async_copy(y_hbm.at[idx_s.at[j]], cur, csem).wait()
                pltpu.sync_copy(cur, agg_sh.at[idx_d.at[j]], add=True)
        plsc.subcore_barrier()

        for k in range(ROWS_PER_TILE // K):
            pltpu.sync_copy(agg_sh.at[pl.ds(base + k * K, K)], rows0)
            pltpu.sync_copy(
                rows0, agg_hbm.at[pl.ds(c * N_PAD + base + k * K, K)])

    return pl.kernel(
        body,
        out_type=jax.ShapeDtypeStruct((NC * N_PAD, D), jnp.float32),
        mesh=mesh,
        scratch_types=[
            pltpu.VMEM((IB, K), jnp.int32),
            pltpu.VMEM((IB, K), jnp.int32),
            pltpu.VMEM((K, D), jnp.float32),
            pltpu.VMEM((K, D), jnp.float32),
            pltpu.VMEM_SHARED((N_PAD, D), jnp.float32),
            pltpu.SemaphoreType.DMA,
            pltpu.SemaphoreType.DMA,
        ],
        name="sc_segment_sum",
    )


def _sc_deg_kernel(ch):
    mesh = plsc.VectorSubcoreMesh(core_axis_name="c", subcore_axis_name="s")

    def body(dsts_hbm, degw_hbm, idx_d, rows, degw_sh):
        c = lax.axis_index("c")
        s = lax.axis_index("s")
        wid = c * NS + s
        base = s * ROWS_PER_TILE

        @pl.loop(0, K)
        def _zfill(i):
            for cc in range(D // 16):
                rows[i, pl.ds(cc * 16, 16)] = jnp.zeros((16,), jnp.float32)
        for k in range(ROWS_PER_TILE // K):
            pltpu.sync_copy(rows, degw_sh.at[pl.ds(base + k * K, K)])

        @pl.loop(0, K)
        def _ofill(i):
            for cc in range(D // 16):
                rows[i, pl.ds(cc * 16, 16)] = jnp.ones((16,), jnp.float32)
        plsc.subcore_barrier()

        @pl.loop(0, ch // IB)
        def _outer(o):
            pltpu.sync_copy(dsts_hbm.at[wid, pl.ds(o * IB, IB)], idx_d)
            for j in range(IB):
                pltpu.sync_copy(rows, degw_sh.at[idx_d.at[j]], add=True)
        plsc.subcore_barrier()

        for k in range(ROWS_PER_TILE // K):
            pltpu.sync_copy(degw_sh.at[pl.ds(base + k * K, K)], rows)
            pltpu.sync_copy(
                rows, degw_hbm.at[pl.ds(c * N_PAD + base + k * K, K)])

    return pl.kernel(
        body,
        out_type=jax.ShapeDtypeStruct((NC * N_PAD, D), jnp.float32),
        mesh=mesh,
        scratch_types=[
            pltpu.VMEM((IB, K), jnp.int32),
            pltpu.VMEM((K, D), jnp.float32),
            pltpu.VMEM_SHARED((N_PAD, D), jnp.float32),
        ],
        name="sc_degree_count",
    )


def _transform_body(h_ref, wl_ref, wr_ref, b_ref, y_ref, z_ref):
    hb = h_ref[...]
    y_ref[...] = jnp.dot(hb, wl_ref[...], preferred_element_type=jnp.float32)
    z_ref[...] = (
        jnp.dot(hb, wr_ref[...], preferred_element_type=jnp.float32)
        + b_ref[...]
    )


def _transform(h, wl, wr, b):
    r = 2000
    return pl.pallas_call(
        _transform_body,
        grid=(N // r,),
        in_specs=[
            pl.BlockSpec((r, D), lambda i: (i, 0)),
            pl.BlockSpec((D, D), lambda i: (0, 0)),
            pl.BlockSpec((D, D), lambda i: (0, 0)),
            pl.BlockSpec((1, D), lambda i: (0, 0)),
        ],
        out_specs=[
            pl.BlockSpec((r, D), lambda i: (i, 0)),
            pl.BlockSpec((r, D), lambda i: (i, 0)),
        ],
        out_shape=[jax.ShapeDtypeStruct((N, D), jnp.float32)] * 2,
    )(h, wl, wr, b.reshape(1, D))


def _combine_body(a_ref, dw_ref, z_ref, o_ref, *, relu):
    deg = dw_ref[0, :, 0:1] + dw_ref[1, :, 0:1]
    recip = 1.0 / jnp.maximum(deg, 1.0)
    o = (a_ref[0] + a_ref[1]) * recip + z_ref[...]
    o_ref[...] = jnp.maximum(o, 0.0) if relu else o


def _combine(agg, degw, z, relu):
    r = 1000
    a3 = agg.reshape(NC, N_PAD, D)
    d3 = degw.reshape(NC, N_PAD, D)
    return pl.pallas_call(
        functools.partial(_combine_body, relu=relu),
        grid=(N // r,),
        in_specs=[
            pl.BlockSpec((NC, r, D), lambda i: (0, i, 0)),
            pl.BlockSpec((NC, r, D), lambda i: (0, i, 0)),
            pl.BlockSpec((r, D), lambda i: (i, 0)),
        ],
        out_specs=pl.BlockSpec((r, D), lambda i: (i, 0)),
        out_shape=jax.ShapeDtypeStruct((N, D), jnp.float32),
    )(a3, d3, z)


def kernel(x, edge_index, W1l, b1, W1r, W2l, b2, W2r, W3l, b3, W3r):
    e = edge_index.shape[1]
    ch = -(-e // (NW * K))
    ch = -(-ch // IB) * IB
    e_pad = NW * ch * K
    src = edge_index[0].astype(jnp.int32)
    dst = edge_index[1].astype(jnp.int32)
    pad = e_pad - e
    pad_src = jnp.arange(pad, dtype=jnp.int32) % N
    pad_dst = N + jnp.arange(pad, dtype=jnp.int32) % (N_PAD - N)
    srcs = jnp.concatenate([src, pad_src])
    dsts = jnp.concatenate([dst, pad_dst])
    srcs = srcs.reshape(NW, ch, K)
    dsts = dsts.reshape(NW, ch, K)

    agg_k = _sc_agg_kernel(ch)
    degw = _sc_deg_kernel(ch)(dsts)

    y1, z1 = _transform(x, W1l, W1r, b1)
    a1 = agg_k(y1, srcs, dsts)
    h1 = _combine(a1, degw, z1, True)

    y2, z2 = _transform(h1, W2l, W2r, b2)
    a2 = agg_k(y2, srcs, dsts)
    h2 = _combine(a2, degw, z2, True)

    y3, z3 = _transform(h2, W3l, W3r, b3)
    a3 = agg_k(y3, srcs, dsts)
    return _combine(a3, degw, z3, False)
